# Initial kernel scaffold; baseline (speedup 1.0000x reference)
import jax
import jax.numpy as jnp
from jax.experimental import pallas as pl


def kernel(x, W, b):
    raise NotImplementedError("write your pallas kernel here")


# speedup vs baseline: 1.8932x; 1.8932x over previous
import functools

import jax
import jax.numpy as jnp
from jax.experimental import pallas as pl

D_MODEL = 768
NUM_EXPERTS = 64
INV_TEMPERATURE = 10.0
BLOCK_T = 1024


def _router_body(x_ref, w_ref, b_ref, logits_ref, probs_ref, idx_ref):
    x = x_ref[...]
    w = w_ref[...]
    acc = jax.lax.dot_general(
        x, w, (((1,), (1,)), ((), ())), preferred_element_type=jnp.float32
    )
    logits = (acc + b_ref[...]) * INV_TEMPERATURE
    logits_ref[...] = logits

    iota = jax.lax.broadcasted_iota(jnp.int32, logits.shape, 1)
    big = jnp.int32(NUM_EXPERTS)
    neg_inf = jnp.float32(-jnp.inf)

    m1 = jnp.max(logits, axis=1, keepdims=True)
    i1 = jnp.min(jnp.where(logits == m1, iota, big), axis=1, keepdims=True)
    masked = jnp.where(iota == i1, neg_inf, logits)
    m2 = jnp.max(masked, axis=1, keepdims=True)
    i2 = jnp.min(jnp.where(masked == m2, iota, big), axis=1, keepdims=True)

    p1 = 1.0 / (1.0 + jnp.exp(m2 - m1))
    probs_ref[...] = jnp.concatenate([p1, 1.0 - p1], axis=1)
    idx_ref[...] = jnp.concatenate([i1, i2], axis=1)


@jax.jit
def kernel(x, W, b):
    n_tokens = x.shape[0]
    grid = (n_tokens // BLOCK_T,)
    out_shapes = (
        jax.ShapeDtypeStruct((n_tokens, NUM_EXPERTS), jnp.float32),
        jax.ShapeDtypeStruct((n_tokens, 2), jnp.float32),
        jax.ShapeDtypeStruct((n_tokens, 2), jnp.int32),
    )
    logits, probs, idx = pl.pallas_call(
        _router_body,
        grid=grid,
        in_specs=[
            pl.BlockSpec((BLOCK_T, D_MODEL), lambda i: (i, 0)),
            pl.BlockSpec((NUM_EXPERTS, D_MODEL), lambda i: (0, 0)),
            pl.BlockSpec((1, NUM_EXPERTS), lambda i: (0, 0)),
        ],
        out_specs=(
            pl.BlockSpec((BLOCK_T, NUM_EXPERTS), lambda i: (i, 0)),
            pl.BlockSpec((BLOCK_T, 2), lambda i: (i, 0)),
            pl.BlockSpec((BLOCK_T, 2), lambda i: (i, 0)),
        ),
        out_shape=out_shapes,
    )(x, W, b.reshape(1, NUM_EXPERTS))
    return logits, probs, idx


# speedup vs baseline: 2.0932x; 1.1056x over previous
import functools

import jax
import jax.numpy as jnp
from jax.experimental import pallas as pl

D_MODEL = 768
NUM_EXPERTS = 64
INV_TEMPERATURE = 10.0
BLOCK_T = 2048


def _router_body(x_ref, w_ref, b_ref, logits_ref, probs_ref, idx_ref):
    x = x_ref[...]
    w = w_ref[...]
    acc = jax.lax.dot_general(
        x, w, (((1,), (1,)), ((), ())), preferred_element_type=jnp.float32
    )
    logits = (acc + b_ref[...]) * INV_TEMPERATURE
    logits_ref[...] = logits

    iota = jax.lax.broadcasted_iota(jnp.int32, logits.shape, 1)
    big = jnp.int32(NUM_EXPERTS)
    neg_inf = jnp.float32(-jnp.inf)

    m1 = jnp.max(logits, axis=1, keepdims=True)
    i1 = jnp.min(jnp.where(logits == m1, iota, big), axis=1, keepdims=True)
    masked = jnp.where(iota == i1, neg_inf, logits)
    m2 = jnp.max(masked, axis=1, keepdims=True)
    i2 = jnp.min(jnp.where(masked == m2, iota, big), axis=1, keepdims=True)

    p1 = 1.0 / (1.0 + jnp.exp(m2 - m1))
    probs_ref[...] = jnp.concatenate([p1, 1.0 - p1], axis=1)
    idx_ref[...] = jnp.concatenate([i1, i2], axis=1)


@jax.jit
def kernel(x, W, b):
    n_tokens = x.shape[0]
    grid = (n_tokens // BLOCK_T,)
    out_shapes = (
        jax.ShapeDtypeStruct((n_tokens, NUM_EXPERTS), jnp.float32),
        jax.ShapeDtypeStruct((n_tokens, 2), jnp.float32),
        jax.ShapeDtypeStruct((n_tokens, 2), jnp.int32),
    )
    logits, probs, idx = pl.pallas_call(
        _router_body,
        grid=grid,
        in_specs=[
            pl.BlockSpec((BLOCK_T, D_MODEL), lambda i: (i, 0)),
            pl.BlockSpec((NUM_EXPERTS, D_MODEL), lambda i: (0, 0)),
            pl.BlockSpec((1, NUM_EXPERTS), lambda i: (0, 0)),
        ],
        out_specs=(
            pl.BlockSpec((BLOCK_T, NUM_EXPERTS), lambda i: (i, 0)),
            pl.BlockSpec((BLOCK_T, 2), lambda i: (i, 0)),
            pl.BlockSpec((BLOCK_T, 2), lambda i: (i, 0)),
        ),
        out_shape=out_shapes,
    )(x, W, b.reshape(1, NUM_EXPERTS))
    return logits, probs, idx


# speedup vs baseline: 2.1859x; 1.0443x over previous
import functools

import jax
import jax.numpy as jnp
from jax.experimental import pallas as pl

D_MODEL = 768
NUM_EXPERTS = 64
INV_TEMPERATURE = 10.0
BLOCK_T = 4096


def _router_body(x_ref, w_ref, b_ref, logits_ref, probs_ref, idx_ref):
    x = x_ref[...]
    w = w_ref[...]
    acc = jax.lax.dot_general(
        x, w, (((1,), (1,)), ((), ())), preferred_element_type=jnp.float32
    )
    logits = (acc + b_ref[...]) * INV_TEMPERATURE
    logits_ref[...] = logits

    iota = jax.lax.broadcasted_iota(jnp.int32, logits.shape, 1)
    big = jnp.int32(NUM_EXPERTS)
    neg_inf = jnp.float32(-jnp.inf)

    m1 = jnp.max(logits, axis=1, keepdims=True)
    i1 = jnp.min(jnp.where(logits == m1, iota, big), axis=1, keepdims=True)
    masked = jnp.where(iota == i1, neg_inf, logits)
    m2 = jnp.max(masked, axis=1, keepdims=True)
    i2 = jnp.min(jnp.where(masked == m2, iota, big), axis=1, keepdims=True)

    p1 = 1.0 / (1.0 + jnp.exp(m2 - m1))
    probs_ref[...] = jnp.concatenate([p1, 1.0 - p1], axis=1)
    idx_ref[...] = jnp.concatenate([i1, i2], axis=1)


@jax.jit
def kernel(x, W, b):
    n_tokens = x.shape[0]
    grid = (n_tokens // BLOCK_T,)
    out_shapes = (
        jax.ShapeDtypeStruct((n_tokens, NUM_EXPERTS), jnp.float32),
        jax.ShapeDtypeStruct((n_tokens, 2), jnp.float32),
        jax.ShapeDtypeStruct((n_tokens, 2), jnp.int32),
    )
    logits, probs, idx = pl.pallas_call(
        _router_body,
        grid=grid,
        in_specs=[
            pl.BlockSpec((BLOCK_T, D_MODEL), lambda i: (i, 0)),
            pl.BlockSpec((NUM_EXPERTS, D_MODEL), lambda i: (0, 0)),
            pl.BlockSpec((1, NUM_EXPERTS), lambda i: (0, 0)),
        ],
        out_specs=(
            pl.BlockSpec((BLOCK_T, NUM_EXPERTS), lambda i: (i, 0)),
            pl.BlockSpec((BLOCK_T, 2), lambda i: (i, 0)),
            pl.BlockSpec((BLOCK_T, 2), lambda i: (i, 0)),
        ),
        out_shape=out_shapes,
    )(x, W, b.reshape(1, NUM_EXPERTS))
    return logits, probs, idx
